# Initial kernel scaffold; baseline (speedup 1.0000x reference)
import functools

import jax
import jax.numpy as jnp
from jax import lax
from jax.experimental import pallas as pl
from jax.experimental.pallas import tpu as pltpu
from jax.experimental.pallas import tpu_sc as plsc

L = 512
D = 16
TABLE_ROWS = 257
REV_ROWS = 1024
NUM_CORES = 2
NUM_SUBCORES = 16
NW = NUM_CORES * NUM_SUBCORES
I_PER_W = L // NW
ROW_WORDS = L * D


def _sc_body(table_hbm, g_hbm, out_hbm, table_v, gbuf_v, rev_v, sem):
    cid = lax.axis_index("c")
    sid = lax.axis_index("s")
    wid = sid * NUM_CORES + cid
    i0 = wid * I_PER_W

    pltpu.sync_copy(table_hbm, table_v)
    pltpu.sync_copy(g_hbm.at[pl.ds((L - 1) * L, L)], gbuf_v.at[pl.ds(0, L)])
    pltpu.sync_copy(g_hbm.at[pl.ds(0, L)], gbuf_v.at[pl.ds(L, L)])

    lane = lax.iota(jnp.int32, D)

    def build_chunk(c, carry):
        n = c * D + lane
        sel = n + jnp.where(n >= L, 1, 0)
        sel = jnp.minimum(sel, 2 * L - 1)
        rowid = plsc.load_gather(gbuf_v, [sel])
        src = rowid * D
        dst = n * D
        for l in range(D):
            vals = plsc.load_gather(table_v, [src + l])
            plsc.store_scatter(rev_v, [dst + l], vals)
        return carry

    lax.fori_loop(31 - wid, 64 - wid, build_chunk, 0)

    copies = []
    for r in range(I_PER_W):
        i = i0 + r
        src_off = pl.multiple_of((L - 1 - i) * D, D)
        dst_off = pl.multiple_of(i * ROW_WORDS, ROW_WORDS)
        copies.append(
            pltpu.async_copy(
                rev_v.at[pl.ds(src_off, ROW_WORDS)],
                out_hbm.at[pl.ds(dst_off, ROW_WORDS)],
                sem,
            )
        )
    for cp in copies:
        cp.wait()


@jax.jit
def _rel_pos_emb(table_flat, gather_indices):
    mesh = plsc.VectorSubcoreMesh(
        core_axis_name="c", subcore_axis_name="s",
        num_cores=NUM_CORES, num_subcores=NUM_SUBCORES,
    )
    return pl.kernel(
        _sc_body,
        out_type=jax.ShapeDtypeStruct((L * L * D,), jnp.float32),
        mesh=mesh,
        scratch_types=[
            pltpu.VMEM((TABLE_ROWS * D,), jnp.float32),
            pltpu.VMEM((2 * L,), jnp.int32),
            pltpu.VMEM((REV_ROWS * D,), jnp.float32),
            pltpu.SemaphoreType.DMA,
        ],
    )(table_flat, gather_indices)


def kernel(rel_pos_bias, gather_indices):
    out_flat = _rel_pos_emb(rel_pos_bias.reshape(-1), gather_indices)
    return out_flat.reshape(L, L, 1, D)


# speedup vs baseline: 6.9883x; 6.9883x over previous
import functools

import jax
import jax.numpy as jnp
from jax import lax
from jax.experimental import pallas as pl
from jax.experimental.pallas import tpu as pltpu
from jax.experimental.pallas import tpu_sc as plsc

L = 512
D = 16
TABLE_ROWS = 257
REV_ROWS = 1024
NUM_CORES = 2
NUM_SUBCORES = 16
NW = NUM_CORES * NUM_SUBCORES
I_PER_W = L // NW
ROW_WORDS = L * D


def _sc_body(table_hbm, g_hbm, out_hbm, table_v, gbuf_v, rev_v, sem):
    cid = lax.axis_index("c")
    sid = lax.axis_index("s")
    wid = sid * NUM_CORES + cid
    i0 = wid * I_PER_W

    pltpu.sync_copy(table_hbm, table_v)
    pltpu.sync_copy(g_hbm.at[pl.ds((L - 1) * L, L)], gbuf_v.at[pl.ds(0, L)])
    pltpu.sync_copy(g_hbm.at[pl.ds(0, L)], gbuf_v.at[pl.ds(L, L)])

    lane = lax.iota(jnp.int32, D)

    def build_chunk(c, carry):
        n = c * D + lane
        sel = n + jnp.where(n >= L, 1, 0)
        sel = jnp.minimum(sel, 2 * L - 1)
        rowid = plsc.load_gather(gbuf_v, [sel])
        src = rowid * D
        dst = n * D
        for l in range(D):
            vals = plsc.load_gather(table_v, [src + l])
            plsc.store_scatter(rev_v, [dst + l], vals)
        return carry

    lax.fori_loop(31 - wid, 64 - wid, build_chunk, 0)

    copies = []
    for r in range(I_PER_W):
        i = i0 + r
        src_off = pl.multiple_of((L - 1 - i) * D, D)
        dst_off = pl.multiple_of(i * ROW_WORDS, ROW_WORDS)
        copies.append(
            pltpu.async_copy(
                rev_v.at[pl.ds(src_off, ROW_WORDS)],
                out_hbm.at[pl.ds(dst_off, ROW_WORDS)],
                sem,
            )
        )
    for cp in copies:
        cp.wait()


@jax.jit
def _rel_pos_emb(table_flat, gather_indices):
    mesh = plsc.VectorSubcoreMesh(
        core_axis_name="c", subcore_axis_name="s",
        num_cores=NUM_CORES, num_subcores=NUM_SUBCORES,
    )
    return pl.kernel(
        _sc_body,
        out_type=jax.ShapeDtypeStruct((L * L * D,), jnp.float32),
        mesh=mesh,
        compiler_params=pltpu.CompilerParams(needs_layout_passes=False),
        scratch_types=[
            pltpu.VMEM((TABLE_ROWS * D,), jnp.float32),
            pltpu.VMEM((2 * L,), jnp.int32),
            pltpu.VMEM((REV_ROWS * D,), jnp.float32),
            pltpu.SemaphoreType.DMA,
        ],
    )(table_flat, gather_indices)


def kernel(rel_pos_bias, gather_indices):
    out_flat = _rel_pos_emb(rel_pos_bias.reshape(-1), gather_indices)
    return out_flat.reshape(L, L, 1, D)


# speedup vs baseline: 13.1973x; 1.8885x over previous
import functools

import jax
import jax.numpy as jnp
from jax import lax
from jax.experimental import pallas as pl
from jax.experimental.pallas import tpu as pltpu
from jax.experimental.pallas import tpu_sc as plsc

L = 512
D = 16
TABLE_ROWS = 257
PADW = 1024
NUM_CORES = 2
NUM_SUBCORES = 16
NW = NUM_CORES * NUM_SUBCORES
I_PER_W = L // NW
SLAB = L * D


def _sc_body(table_hbm, g_hbm, out_hbm, table_v, gbuf_v, revt_v, slab_v, sem):
    cid = lax.axis_index("c")
    sid = lax.axis_index("s")
    wid = sid * NUM_CORES + cid
    i0 = wid * I_PER_W

    pltpu.sync_copy(table_hbm, table_v)
    pltpu.sync_copy(g_hbm.at[pl.ds((L - 1) * L, L)], gbuf_v.at[pl.ds(0, L)])
    pltpu.sync_copy(g_hbm.at[pl.ds(0, L)], gbuf_v.at[pl.ds(L, L)])

    lane = lax.iota(jnp.int32, D)

    def build_chunk(c, carry):
        n = c * D + lane
        sel = n + jnp.where(n >= L, 1, 0)
        sel = jnp.minimum(sel, 2 * L - 1)
        rowid = plsc.load_gather(gbuf_v, [sel])
        src = rowid * D
        for h in range(D):
            vals = plsc.load_gather(table_v, [src + h])
            revt_v[pl.ds(h * PADW + c * D, D)] = vals
        return carry

    lax.fori_loop(31 - wid, 64 - wid, build_chunk, 0)

    descs = [None, None]
    for r in range(I_PER_W):
        b = r % 2
        if descs[b] is not None:
            descs[b].wait()
        i = i0 + r
        base = (L - 1) - i

        def asm_row(t, carry):
            dst_row = (t // 8) * (4 * 1024) + (t % 8) * 128 + b * SLAB
            src_row = t * PADW + base
            for jt in range(4):
                for k in range(8):
                    off = jt * 128 + k * D
                    vals = plsc.load_gather(revt_v, [src_row + off + lane])
                    slab_v[pl.ds(dst_row + jt * 1024 + k * D, D)] = vals
            return carry

        lax.fori_loop(0, D, asm_row, 0)
        descs[b] = pltpu.async_copy(
            slab_v.at[pl.ds(b * SLAB, SLAB)],
            out_hbm.at[pl.ds(i * SLAB, SLAB)],
            sem,
        )
    for d in descs:
        d.wait()


@jax.jit
def _rel_pos_emb(table_flat, gather_indices):
    mesh = plsc.VectorSubcoreMesh(
        core_axis_name="c", subcore_axis_name="s",
        num_cores=NUM_CORES, num_subcores=NUM_SUBCORES,
    )
    return pl.kernel(
        _sc_body,
        out_type=jax.ShapeDtypeStruct((L * L * D,), jnp.float32),
        mesh=mesh,
        compiler_params=pltpu.CompilerParams(needs_layout_passes=False),
        scratch_types=[
            pltpu.VMEM((TABLE_ROWS * D,), jnp.float32),
            pltpu.VMEM((2 * L,), jnp.int32),
            pltpu.VMEM((D * PADW,), jnp.float32),
            pltpu.VMEM((2 * SLAB,), jnp.float32),
            pltpu.SemaphoreType.DMA,
        ],
    )(table_flat, gather_indices)


def kernel(rel_pos_bias, gather_indices):
    x = _rel_pos_emb(rel_pos_bias.reshape(-1), gather_indices)
    x = x.reshape(L, 2, 4, 8, 128)
    x = x.transpose(0, 1, 3, 2, 4).reshape(L, D, L)
    return x.transpose(0, 2, 1).reshape(L, L, 1, D)


# speedup vs baseline: 23.6720x; 1.7937x over previous
import functools

import jax
import jax.numpy as jnp
from jax import lax
from jax.experimental import pallas as pl
from jax.experimental.pallas import tpu as pltpu
from jax.experimental.pallas import tpu_sc as plsc

L = 512
D = 16
TABLE_ROWS = 257
PADW = 1024
NUM_CORES = 2
NUM_SUBCORES = 16
NW = NUM_CORES * NUM_SUBCORES
I_PER_W = L // NW
SLAB = L * D


def _sc_body(table_hbm, g_hbm, out_hbm, table_v, gbuf_v, revt_v, slab_v, sem):
    cid = lax.axis_index("c")
    sid = lax.axis_index("s")
    wid = sid * NUM_CORES + cid
    i0 = wid * I_PER_W

    pltpu.sync_copy(table_hbm, table_v)
    pltpu.sync_copy(g_hbm.at[pl.ds((L - 1) * L, L)], gbuf_v.at[pl.ds(0, L)])
    pltpu.sync_copy(g_hbm.at[pl.ds(0, L)], gbuf_v.at[pl.ds(L, L)])

    lane = lax.iota(jnp.int32, D)

    def build_chunk(c, carry):
        n = c * D + lane
        sel = n + jnp.where(n >= L, 1, 0)
        sel = jnp.minimum(sel, 2 * L - 1)
        rowid = plsc.load_gather(gbuf_v, [sel])
        src = rowid * D
        for h in range(D):
            vals = plsc.load_gather(table_v, [src + h])
            revt_v[pl.ds(h * PADW + c * D, D)] = vals
        return carry

    lax.fori_loop(31 - wid, 64 - wid, build_chunk, 0)

    descs = [None, None]
    for r in range(I_PER_W):
        b = r % 2
        if descs[b] is not None:
            descs[b].wait()
        i = i0 + r
        base = (L - 1) - i

        def asm_row(t, carry):
            dst_row = (t // 8) * (4 * 1024) + (t % 8) * 128 + b * SLAB
            src_row = t * PADW + base
            for half in range(2):
                vals = []
                for c in range(D):
                    off = half * 256 + c * D
                    vals.append(plsc.load_gather(revt_v, [src_row + off + lane]))
                for c in range(D):
                    off = half * 256 + c * D
                    jt, k = off // 128, (off % 128) // D
                    slab_v[pl.ds(dst_row + jt * 1024 + k * D, D)] = vals[c]
            return carry

        lax.fori_loop(0, D, asm_row, 0)
        descs[b] = pltpu.async_copy(
            slab_v.at[pl.ds(b * SLAB, SLAB)],
            out_hbm.at[pl.ds(i * SLAB, SLAB)],
            sem,
        )
    for d in descs:
        d.wait()


@jax.jit
def _rel_pos_emb(table_flat, gather_indices):
    mesh = plsc.VectorSubcoreMesh(
        core_axis_name="c", subcore_axis_name="s",
        num_cores=NUM_CORES, num_subcores=NUM_SUBCORES,
    )
    return pl.kernel(
        _sc_body,
        out_type=jax.ShapeDtypeStruct((L * L * D,), jnp.float32),
        mesh=mesh,
        compiler_params=pltpu.CompilerParams(needs_layout_passes=False),
        scratch_types=[
            pltpu.VMEM((TABLE_ROWS * D,), jnp.float32),
            pltpu.VMEM((2 * L,), jnp.int32),
            pltpu.VMEM((D * PADW,), jnp.float32),
            pltpu.VMEM((2 * SLAB,), jnp.float32),
            pltpu.SemaphoreType.DMA,
        ],
    )(table_flat, gather_indices)


def kernel(rel_pos_bias, gather_indices):
    x = _rel_pos_emb(rel_pos_bias.reshape(-1), gather_indices)
    x = x.reshape(L, 2, 4, 8, 128)
    x = x.transpose(0, 1, 3, 2, 4).reshape(L, D, L)
    return x.transpose(0, 2, 1).reshape(L, L, 1, D)
